# baseline (speedup 1.0000x reference)
import functools

import jax
import jax.numpy as jnp
import numpy as np
from jax.experimental import pallas as pl
from jax.experimental.pallas import tpu as pltpu


_TOPK_RATIO = 0.1


def _attn_block_kernel(q_ref, k_ref, v_ref, o_ref, s_scr, key_scr, chop_scr,
                       *, k_sel, scale):
    int_min = jnp.int32(-2147483648)
    q = q_ref[0]
    k = k_ref[0]
    v = v_ref[0]
    s = jax.lax.dot_general(q, k, (((1,), (1,)), ((), ())),
                            preferred_element_type=jnp.float32) * scale
    s_scr[...] = s
    ikeys = jax.lax.bitcast_convert_type(s, jnp.int32)
    mono = ikeys ^ (jax.lax.shift_right_arithmetic(ikeys, 31)
                    & jnp.int32(0x7FFFFFFF))
    key_scr[...] = mono
    chop_scr[...] = jax.lax.bitcast_convert_type(
        jax.lax.shift_right_logical(ikeys, 16).astype(jnp.int16),
        jnp.bfloat16)

    bq, ss = s.shape
    kc = jnp.int32(k_sel)
    kf = jnp.float32(k_sel)
    one_bf = jnp.bfloat16(1.0)
    zero_bf = jnp.bfloat16(0.0)
    ncol = ss // 128

    def body_hi(i, t):
        bit = jnp.left_shift(jnp.int32(1), 15 - i)
        cand_u = t | bit
        m = cand_u ^ jnp.int32(0x8000)
        flip = (jax.lax.shift_right_logical(m, 15) & jnp.int32(1)) * \
            jnp.int32(0x7FFF)
        cand_bf = jax.lax.bitcast_convert_type(
            (m ^ flip).astype(jnp.int16), jnp.bfloat16)
        onz = jnp.where(chop_scr[...] >= cand_bf, one_bf, zero_bf)
        acc = onz[:, :128]
        for j in range(1, ncol):
            acc = acc + onz[:, j * 128:(j + 1) * 128]
        w = 64
        while w >= 8:
            acc = acc[:, :w] + acc[:, w:]
            w //= 2
        cnt = jnp.sum(acc.astype(jnp.float32), axis=1, keepdims=True)
        return jnp.where(cnt >= kf, cand_u, t)

    def body_lo(i, t):
        bit = jnp.left_shift(jnp.int32(1), 31 - i)
        cand_u = t | bit
        cand_s = cand_u ^ int_min
        cnt = jnp.sum((key_scr[...] >= cand_s).astype(jnp.int32), axis=1,
                      keepdims=True)
        return jnp.where(cnt >= kc, cand_u, t)

    t0 = jnp.zeros((bq, 1), jnp.int32)
    t_hi = jax.lax.fori_loop(0, 16, body_hi, t0, unroll=8)
    t32 = jnp.left_shift(t_hi, 16)
    t = jax.lax.fori_loop(16, 24, body_lo, t32, unroll=8)

    thresh_s = t ^ int_min
    thresh_i = thresh_s ^ (jax.lax.shift_right_arithmetic(thresh_s, 31)
                           & jnp.int32(0x7FFFFFFF))
    thresh_f = jax.lax.bitcast_convert_type(thresh_i, jnp.float32)

    s = s_scr[...]
    neg = jnp.finfo(jnp.float32).min
    masked = jnp.where(s >= thresh_f, s, neg)
    m = jnp.max(masked, axis=1, keepdims=True)
    e = jnp.exp(masked - m)
    denom = jnp.sum(e, axis=1, keepdims=True)
    p = (e / denom).astype(jnp.bfloat16)
    o = jax.lax.dot_general(p, v.astype(jnp.bfloat16),
                            (((1,), (0,)), ((), ())),
                            preferred_element_type=jnp.float32)
    o_ref[0] = o


def kernel(query, key, value):
    B, S, H, D = query.shape
    assert B == 1
    k_sel = max(1, int(_TOPK_RATIO * S))
    scale = 1.0 / float(np.sqrt(D))
    BQ = 512
    while S % BQ:
        BQ //= 2
    NQ = S // BQ

    q3 = query[0].transpose(1, 0, 2)
    k3 = key[0].transpose(1, 0, 2)
    v3 = value[0].transpose(1, 0, 2)

    grid = (H, NQ)
    out = pl.pallas_call(
        functools.partial(_attn_block_kernel, k_sel=k_sel, scale=scale),
        grid=grid,
        in_specs=[
            pl.BlockSpec((1, BQ, D), lambda h, qb: (h, qb, 0)),
            pl.BlockSpec((1, S, D), lambda h, qb: (h, 0, 0)),
            pl.BlockSpec((1, S, D), lambda h, qb: (h, 0, 0)),
        ],
        out_specs=pl.BlockSpec((1, BQ, D), lambda h, qb: (0, qb, h)),
        out_shape=jax.ShapeDtypeStruct((1, S, H * D), jnp.float32),
        scratch_shapes=[
            pltpu.VMEM((BQ, S), jnp.float32),
            pltpu.VMEM((BQ, S), jnp.int32),
            pltpu.VMEM((BQ, S), jnp.bfloat16),
        ],
    )(q3, k3, v3)
    return out


# speedup vs baseline: 1.3536x; 1.3536x over previous
import functools

import jax
import jax.numpy as jnp
import numpy as np
from jax.experimental import pallas as pl
from jax.experimental.pallas import tpu as pltpu


_TOPK_RATIO = 0.1


def _attn_block_kernel(q_ref, k_ref, v_ref, o_ref, s_scr, key_scr, chop_scr,
                       *, k_sel, scale):
    int_min = jnp.int32(-2147483648)
    q = q_ref[0]
    k = k_ref[0]
    v = v_ref[0]
    s = jax.lax.dot_general(q, k, (((1,), (1,)), ((), ())),
                            preferred_element_type=jnp.float32) * scale
    s_scr[...] = s
    ikeys = jax.lax.bitcast_convert_type(s, jnp.int32)
    mono = ikeys ^ (jax.lax.shift_right_arithmetic(ikeys, 31)
                    & jnp.int32(0x7FFFFFFF))
    key_scr[...] = mono
    chop_scr[...] = jax.lax.bitcast_convert_type(
        jax.lax.shift_right_logical(ikeys, 16).astype(jnp.int16),
        jnp.bfloat16)

    bq, ss = s.shape
    kc = jnp.int32(k_sel)
    kf = jnp.float32(k_sel)
    one_bf = jnp.bfloat16(1.0)
    zero_bf = jnp.bfloat16(0.0)
    ncol = ss // 128

    def body_hi(i, t):
        bit = jnp.left_shift(jnp.int32(1), 15 - i)
        cand_u = t | bit
        m = cand_u ^ jnp.int32(0x8000)
        flip = (jax.lax.shift_right_logical(m, 15) & jnp.int32(1)) * \
            jnp.int32(0x7FFF)
        cand_bf = jax.lax.bitcast_convert_type(
            (m ^ flip).astype(jnp.int16), jnp.bfloat16)
        onz = jnp.where(chop_scr[...] >= cand_bf, one_bf, zero_bf)
        acc = onz[:, :128]
        for j in range(1, ncol):
            acc = acc + onz[:, j * 128:(j + 1) * 128]
        cnt = jnp.sum(acc.astype(jnp.float32), axis=1, keepdims=True)
        return jnp.where(cnt >= kf, cand_u, t)

    def body_lo(i, t):
        bit = jnp.left_shift(jnp.int32(1), 31 - i)
        cand_u = t | bit
        cand_s = cand_u ^ int_min
        cnt = jnp.sum((key_scr[...] >= cand_s).astype(jnp.int32), axis=1,
                      keepdims=True)
        return jnp.where(cnt >= kc, cand_u, t)

    t0 = jnp.zeros((bq, 1), jnp.int32)
    t_hi = jax.lax.fori_loop(0, 16, body_hi, t0, unroll=16)
    t32 = jnp.left_shift(t_hi, 16)
    t = jax.lax.fori_loop(16, 24, body_lo, t32, unroll=8)

    thresh_s = t ^ int_min
    thresh_i = thresh_s ^ (jax.lax.shift_right_arithmetic(thresh_s, 31)
                           & jnp.int32(0x7FFFFFFF))
    thresh_f = jax.lax.bitcast_convert_type(thresh_i, jnp.float32)

    s = s_scr[...]
    neg = jnp.finfo(jnp.float32).min
    masked = jnp.where(s >= thresh_f, s, neg)
    m = jnp.max(masked, axis=1, keepdims=True)
    e = jnp.exp(masked - m)
    denom = jnp.sum(e, axis=1, keepdims=True)
    p = (e / denom).astype(jnp.bfloat16)
    o = jax.lax.dot_general(p, v.astype(jnp.bfloat16),
                            (((1,), (0,)), ((), ())),
                            preferred_element_type=jnp.float32)
    o_ref[0] = o


def kernel(query, key, value):
    B, S, H, D = query.shape
    assert B == 1
    k_sel = max(1, int(_TOPK_RATIO * S))
    scale = 1.0 / float(np.sqrt(D))
    BQ = 1024
    while S % BQ:
        BQ //= 2
    NQ = S // BQ

    q3 = query[0].transpose(1, 0, 2)
    k3 = key[0].transpose(1, 0, 2)
    v3 = value[0].transpose(1, 0, 2)

    grid = (H, NQ)
    out = pl.pallas_call(
        functools.partial(_attn_block_kernel, k_sel=k_sel, scale=scale),
        grid=grid,
        in_specs=[
            pl.BlockSpec((1, BQ, D), lambda h, qb: (h, qb, 0)),
            pl.BlockSpec((1, S, D), lambda h, qb: (h, 0, 0)),
            pl.BlockSpec((1, S, D), lambda h, qb: (h, 0, 0)),
        ],
        out_specs=pl.BlockSpec((1, BQ, D), lambda h, qb: (0, qb, h)),
        out_shape=jax.ShapeDtypeStruct((1, S, H * D), jnp.float32),
        scratch_shapes=[
            pltpu.VMEM((BQ, S), jnp.float32),
            pltpu.VMEM((BQ, S), jnp.int32),
            pltpu.VMEM((BQ, S), jnp.bfloat16),
        ],
    )(q3, k3, v3)
    return out
